# Initial kernel scaffold; baseline (speedup 1.0000x reference)
import jax
import jax.numpy as jnp
from jax.experimental import pallas as pl


def kernel(x, token_table, pos_table):
    raise NotImplementedError("write your pallas kernel here")


# speedup vs baseline: 2.4114x; 2.4114x over previous
import functools

import jax
import jax.numpy as jnp
from jax import lax
from jax.experimental import pallas as pl
from jax.experimental.pallas import tpu as pltpu
from jax.experimental.pallas import tpu_sc as plsc

D = 128
CH = 100
NC = 2
NS = 16
NW = NC * NS


@functools.lru_cache(maxsize=None)
def _build(nchunk):
    cpw = nchunk // NW
    mesh = plsc.VectorSubcoreMesh(core_axis_name="c", subcore_axis_name="s")

    @functools.partial(
        pl.kernel,
        mesh=mesh,
        out_type=jax.ShapeDtypeStruct((nchunk, CH, D), jnp.float32),
        scratch_types=[
            pltpu.VMEM((CH,), jnp.int32),
            pltpu.VMEM((CH, D), jnp.float32),
            pltpu.VMEM((CH, D), jnp.float32),
            pltpu.SemaphoreType.DMA,
        ],
    )
    def emb(x_hbm, tok_hbm, pos_hbm, out_hbm, idx_v, rows_v, pos_v, sem):
        wid = lax.axis_index("s") * NC + lax.axis_index("c")
        parity = wid % 2
        pltpu.sync_copy(pos_hbm.at[parity], pos_v)

        def chunk(j, carry):
            g = wid + NW * j
            pltpu.sync_copy(x_hbm.at[g], idx_v)
            pltpu.async_copy(tok_hbm.at[idx_v], rows_v, sem).wait()

            def row(i, c):
                for d in range(D // 16):
                    s0 = pl.ds(d * 16, 16)
                    rows_v[i, s0] = rows_v[i, s0] + pos_v[i, s0]
                return c

            lax.fori_loop(0, CH, row, 0)
            pltpu.sync_copy(rows_v, out_hbm.at[g])
            return carry

        lax.fori_loop(0, cpw, chunk, 0)

    return emb


def kernel(x, token_table, pos_table):
    B, T = x.shape
    nchunk = (B * T) // CH
    x2 = x.reshape(nchunk, CH).astype(jnp.int32)
    pos2 = pos_table.reshape(T // CH, CH, D)
    out = _build(nchunk)(x2, token_table, pos2)
    return out.reshape(B, T, D)


# speedup vs baseline: 2.6188x; 1.0860x over previous
import functools

import jax
import jax.numpy as jnp
from jax import lax
from jax.experimental import pallas as pl
from jax.experimental.pallas import tpu as pltpu
from jax.experimental.pallas import tpu_sc as plsc

D = 128
CH = 100
G = 2
NC = 2
NS = 16
NW = NC * NS


@functools.lru_cache(maxsize=None)
def _build(nchunk):
    cpw = nchunk // NW
    ng = cpw // G
    mesh = plsc.VectorSubcoreMesh(core_axis_name="c", subcore_axis_name="s")

    @functools.partial(
        pl.kernel,
        mesh=mesh,
        out_type=jax.ShapeDtypeStruct((nchunk, CH, D), jnp.float32),
        scratch_types=[
            pltpu.VMEM((cpw, CH), jnp.int32),
            pltpu.VMEM((2, G, CH, D), jnp.float32),
            pltpu.VMEM((2, CH, D), jnp.float32),
            pltpu.SemaphoreType.DMA,
            pltpu.SemaphoreType.DMA,
            pltpu.SemaphoreType.DMA,
            pltpu.SemaphoreType.DMA,
        ],
    )
    def emb(x_hbm, tok_hbm, pos_hbm, out_hbm, idx_v, rows_v, pos_v,
            sg0, sg1, so0, so1):
        wid = lax.axis_index("s") * NC + lax.axis_index("c")
        base = wid * cpw
        sgs = (sg0, sg1)
        sos = (so0, so1)

        pltpu.sync_copy(pos_hbm, pos_v)
        pltpu.sync_copy(x_hbm.at[pl.ds(base, cpw)], idx_v)

        def start_group(jj, b):
            for c in range(G):
                pltpu.async_copy(tok_hbm.at[idx_v.at[jj * G + c]],
                                 rows_v.at[b].at[c], sgs[b])

        def wait_gathers(b):
            for c in range(G):
                pltpu.make_async_copy(tok_hbm.at[idx_v.at[c]],
                                      rows_v.at[b].at[c], sgs[b]).wait()

        def start_out(jj, b):
            pltpu.async_copy(rows_v.at[b],
                             out_hbm.at[pl.ds(base + jj * G, G)], sos[b])

        def wait_out(b):
            pltpu.make_async_copy(rows_v.at[b],
                                  out_hbm.at[pl.ds(base, G)], sos[b]).wait()

        def add_pos(b):
            for c in range(G):
                p = c % 2

                def row(i, carry):
                    for d in range(D // 16):
                        s0 = pl.ds(d * 16, 16)
                        rows_v[b, c, i, s0] = rows_v[b, c, i, s0] + pos_v[p, i, s0]
                    return carry

                lax.fori_loop(0, CH, row, 0, unroll=2)

        start_group(0, 0)

        def body(j2, carry):
            for b in range(2):
                jj = 2 * j2 + b
                nb = 1 - b
                have_next = jj + 1 < ng
                if b == 0:
                    can_wait = jnp.logical_and(have_next, j2 >= 1)
                else:
                    can_wait = have_next

                @pl.when(can_wait)
                def _():
                    wait_out(nb)

                @pl.when(have_next)
                def _():
                    start_group(jj + 1, nb)

                wait_gathers(b)
                add_pos(b)
                start_out(jj, b)
            return carry

        lax.fori_loop(0, ng // 2, body, 0)
        wait_out(0)
        wait_out(1)

    return emb


def kernel(x, token_table, pos_table):
    B, T = x.shape
    nchunk = (B * T) // CH
    x2 = x.reshape(nchunk, CH).astype(jnp.int32)
    pos2 = pos_table.reshape(T // CH, CH, D)
    out = _build(nchunk)(x2, token_table, pos2)
    return out.reshape(B, T, D)


# speedup vs baseline: 3.9867x; 1.5223x over previous
import functools

import jax
import jax.numpy as jnp
from jax import lax
from jax.experimental import pallas as pl
from jax.experimental.pallas import tpu as pltpu
from jax.experimental.pallas import tpu_sc as plsc

D = 128
CH = 100
NC = 2
NS = 16
NW = NC * NS


@functools.lru_cache(maxsize=None)
def _build(B, T):
    rpw = B // NW
    nix = T // CH
    mesh = plsc.VectorSubcoreMesh(core_axis_name="c", subcore_axis_name="s")

    @functools.partial(
        pl.kernel,
        mesh=mesh,
        out_type=jax.ShapeDtypeStruct((B, T, D), jnp.float32),
        scratch_types=[
            pltpu.VMEM((rpw * nix, CH), jnp.int32),
            pltpu.VMEM((2, T, D), jnp.float32),
            pltpu.VMEM((T, D), jnp.float32),
            pltpu.SemaphoreType.DMA,
            pltpu.SemaphoreType.DMA,
            pltpu.SemaphoreType.DMA,
            pltpu.SemaphoreType.DMA,
        ],
    )
    def emb(x_hbm, tok_hbm, pos_hbm, out_hbm, idx_v, rows_v, pos_v,
            sg0, sg1, so0, so1):
        wid = lax.axis_index("s") * NC + lax.axis_index("c")
        base = wid * rpw
        sgs = (sg0, sg1)
        sos = (so0, so1)

        pltpu.sync_copy(pos_hbm, pos_v)
        pltpu.sync_copy(x_hbm.at[pl.ds(base * nix, rpw * nix)], idx_v)

        def start_row(jj, b):
            for c in range(nix):
                pltpu.async_copy(tok_hbm.at[idx_v.at[jj * nix + c]],
                                 rows_v.at[b].at[pl.ds(c * CH, CH)], sgs[b])

        def wait_gathers(b):
            for c in range(nix):
                pltpu.make_async_copy(tok_hbm.at[idx_v.at[c]],
                                      rows_v.at[b].at[pl.ds(c * CH, CH)],
                                      sgs[b]).wait()

        def start_out(jj, b):
            pltpu.async_copy(rows_v.at[b], out_hbm.at[base + jj], sos[b])

        def wait_out(b):
            pltpu.make_async_copy(rows_v.at[b], out_hbm.at[base], sos[b]).wait()

        def add_pos(b):
            def row(i, carry):
                for d in range(D // 16):
                    s0 = pl.ds(d * 16, 16)
                    rows_v[b, i, s0] = rows_v[b, i, s0] + pos_v[i, s0]
                return carry

            lax.fori_loop(0, T, row, 0, unroll=2)

        start_row(0, 0)

        def body(j2, carry):
            for b in range(2):
                jj = 2 * j2 + b
                nb = 1 - b
                have_next = jj + 1 < rpw
                if b == 0:
                    can_wait = jnp.logical_and(have_next, j2 >= 1)
                else:
                    can_wait = have_next

                @pl.when(can_wait)
                def _():
                    wait_out(nb)

                @pl.when(have_next)
                def _():
                    start_row(jj + 1, nb)

                wait_gathers(b)
                add_pos(b)
                start_out(jj, b)
            return carry

        lax.fori_loop(0, rpw // 2, body, 0)
        wait_out(0)
        wait_out(1)

    return emb


def kernel(x, token_table, pos_table):
    B, T = x.shape
    x2 = x.reshape((B * T) // CH, CH).astype(jnp.int32)
    return _build(B, T)(x2, token_table, pos_table)
